# baseline (speedup 1.0000x reference)
import functools

import jax
import jax.numpy as jnp
from jax import lax
from jax.experimental import pallas as pl
from jax.experimental.pallas import tpu as pltpu
from jax.experimental.pallas import tpu_sc as plsc

_NC = 2
_NS = 16
_NW = _NC * _NS
_L = 16
_K = 128


def _project(features, weight, rows_per_block=2000):
    n, f = features.shape
    m = weight.shape[1]

    def body(f_ref, w_ref, o_ref):
        o_ref[...] = jnp.dot(f_ref[...], w_ref[...],
                             preferred_element_type=jnp.float32)

    return pl.pallas_call(
        body,
        grid=(n // rows_per_block,),
        in_specs=[
            pl.BlockSpec((rows_per_block, f), lambda r: (r, 0)),
            pl.BlockSpec((f, m), lambda r: (0, 0)),
        ],
        out_specs=pl.BlockSpec((rows_per_block, m), lambda r: (r, 0)),
        out_shape=jax.ShapeDtypeStruct((n, m), jnp.float32),
    )(features, weight)


def _combine(parts, n, rows_per_block=2000):
    m = parts.shape[2]

    def body(p_ref, o_ref):
        o_ref[...] = p_ref[0] + p_ref[1]

    return pl.pallas_call(
        body,
        grid=(n // rows_per_block,),
        in_specs=[pl.BlockSpec((2, rows_per_block, m), lambda r: (0, r, 0))],
        out_specs=pl.BlockSpec((rows_per_block, m), lambda r: (r, 0)),
        out_shape=jax.ShapeDtypeStruct((n, m), jnp.float32),
    )(parts)


@functools.lru_cache
def _make_spmm(n_out, m, e_pad):
    ept = e_pad // _NW
    n_chunks = ept // _K
    rpt = n_out // _NS
    q_per_row = m // _L
    assert n_chunks % 2 == 0 and n_chunks >= 4

    mesh = plsc.VectorSubcoreMesh(core_axis_name="c", subcore_axis_name="s")

    @functools.partial(
        pl.kernel,
        out_type=jax.ShapeDtypeStruct((2, n_out, m), jnp.float32),
        mesh=mesh,
        scratch_types=[
            pltpu.VMEM((2, 3, _K), jnp.int32),
            pltpu.VMEM((2, _K, m), jnp.float32),
            pltpu.VMEM_SHARED((n_out, m), jnp.float32),
            pltpu.SemaphoreType.DMA,
            pltpu.SemaphoreType.DMA,
            pltpu.SemaphoreType.DMA,
            pltpu.SemaphoreType.DMA,
        ],
        compiler_params=pltpu.CompilerParams(use_tc_tiling_on_sc=False,
                                             needs_layout_passes=False),
    )
    def spmm(x_hbm, edata_hbm, out_hbm,
             ebuf, rows, acc_sh, sem_i0, sem_i1, sem_g0, sem_g1):
        c = lax.axis_index("c")
        s = lax.axis_index("s")
        sem_i = (sem_i0, sem_i1)
        sem_g = (sem_g0, sem_g1)

        def zero_row(i, carry):
            for q in range(q_per_row):
                rows[0, i, pl.ds(q * _L, _L)] = jnp.zeros((_L,), jnp.float32)
            return carry
        lax.fori_loop(0, _K, zero_row, 0)

        row0 = s * rpt
        nfull = rpt // _K
        rem = rpt % _K
        for b in range(nfull):
            pltpu.sync_copy(rows.at[0], acc_sh.at[pl.ds(row0 + b * _K, _K)])
        if rem:
            pltpu.sync_copy(rows.at[0].at[pl.ds(0, rem)],
                            acc_sh.at[pl.ds(row0 + nfull * _K, rem)])
        plsc.subcore_barrier()

        wid = c * _NS + s
        cbase = wid * n_chunks

        def issue_idx(j, p):
            pltpu.async_copy(edata_hbm.at[cbase + j], ebuf.at[p], sem_i[p])

        def wait_idx(p):
            pltpu.make_async_copy(edata_hbm.at[0], ebuf.at[p], sem_i[p]).wait()

        def issue_gather(p):
            pltpu.async_copy(x_hbm.at[ebuf.at[p].at[0]], rows.at[p], sem_g[p])

        def wait_gather(p):
            pltpu.make_async_copy(x_hbm.at[pl.ds(0, _K)], rows.at[p],
                                  sem_g[p]).wait()

        def scale_scatter(p, unrolled):
            rows_p = rows.at[p]

            def group(g, carry):
                av_bits = ebuf[p, 2, pl.ds(g * _L, _L)]
                av = plsc.bitcast(av_bits, jnp.float32)
                e0 = g * _L
                for i in range(_L):
                    a = av[i]
                    for q in range(q_per_row):
                        sl = pl.ds(q * _L, _L)
                        rows_p[e0 + i, sl] = rows_p[e0 + i, sl] * a
                return carry
            if unrolled:
                for g in range(_K // _L):
                    group(g, 0)
            else:
                lax.fori_loop(0, _K // _L, group, 0)
            pltpu.sync_copy(rows_p, acc_sh.at[ebuf.at[p].at[1]], add=True)

        issue_idx(0, 0)
        wait_idx(0)
        issue_gather(0)
        issue_idx(1, 1)

        def pair(jp, carry):
            j = 2 * jp
            wait_idx(1)
            issue_gather(1)
            wait_gather(0)
            scale_scatter(0, True)
            issue_idx(j + 2, 0)
            wait_idx(0)
            issue_gather(0)
            wait_gather(1)
            scale_scatter(1, True)
            issue_idx(j + 3, 1)
            return carry
        lax.fori_loop(0, n_chunks // 2 - 1, pair, 0)

        wait_idx(1)
        issue_gather(1)
        wait_gather(0)
        scale_scatter(0, False)
        wait_gather(1)
        scale_scatter(1, False)

        plsc.subcore_barrier()
        for b in range(nfull):
            sl = pl.ds(row0 + b * _K, _K)
            pltpu.sync_copy(acc_sh.at[sl], out_hbm.at[c].at[sl])
        if rem:
            sl = pl.ds(row0 + nfull * _K, rem)
            pltpu.sync_copy(acc_sh.at[sl], out_hbm.at[c].at[sl])

    return spmm


def kernel(features, adj_values, weight, edge_index, times):
    n, _ = features.shape
    m = weight.shape[1]
    e = edge_index.shape[1]

    src = edge_index[1].astype(jnp.int32)
    dst = edge_index[0].astype(jnp.int32)
    adj = adj_values.astype(jnp.float32)

    grain = _NW * 2 * _K
    e_pad = ((e + grain - 1) // grain) * grain
    if e_pad != e:
        pad = e_pad - e
        src = jnp.concatenate([src, jnp.zeros((pad,), jnp.int32)])
        dst = jnp.concatenate([dst, jnp.zeros((pad,), jnp.int32)])
        adj = jnp.concatenate([adj, jnp.zeros((pad,), jnp.float32)])

    n_chunks = e_pad // (_NW * _K)
    adj_bits = lax.bitcast_convert_type(adj, jnp.int32)
    edata = jnp.stack([src, dst, adj_bits])
    edata = edata.reshape(3, _NW, n_chunks, _K)
    edata = jnp.transpose(edata, (1, 2, 0, 3)).reshape(_NW * n_chunks, 3, _K)

    row_grain = _NS * 8
    n_pad = ((n + row_grain - 1) // row_grain) * row_grain

    x = _project(features, weight)
    spmm = _make_spmm(n_pad, m, e_pad)

    def one_round(xi):
        return _combine(spmm(xi, edata), n)

    x = one_round(x)
    x = lax.fori_loop(1, times, lambda i, o: one_round(o), x)
    return x


# speedup vs baseline: 1.6852x; 1.6852x over previous
import functools

import jax
import jax.numpy as jnp
from jax import lax
from jax.experimental import pallas as pl
from jax.experimental.pallas import tpu as pltpu
from jax.experimental.pallas import tpu_sc as plsc

_NC = 2
_NS = 16
_L = 16
_K = 128


def _project_halves(features, weight, rows_per_block=2000):
    n, f = features.shape
    m = weight.shape[1]
    half = m // 2

    def body(f_ref, w_ref, o_ref):
        o_ref[0] = jnp.dot(f_ref[...], w_ref[0],
                           preferred_element_type=jnp.float32)

    w_halves = jnp.swapaxes(weight.reshape(f, 2, half), 0, 1)
    return pl.pallas_call(
        body,
        grid=(2, n // rows_per_block),
        in_specs=[
            pl.BlockSpec((rows_per_block, f), lambda c, r: (r, 0)),
            pl.BlockSpec((1, f, half), lambda c, r: (c, 0, 0)),
        ],
        out_specs=pl.BlockSpec((1, rows_per_block, half), lambda c, r: (c, r, 0)),
        out_shape=jax.ShapeDtypeStruct((2, n, half), jnp.float32),
    )(features, w_halves)


@functools.lru_cache
def _make_spmm(n_x, n_out, half, e_pad):
    ept = e_pad // _NS
    n_chunks = ept // _K
    rpt = n_out // _NS
    q_per_row = half // _L
    assert n_chunks % 2 == 0 and n_chunks >= 4

    mesh = plsc.VectorSubcoreMesh(core_axis_name="c", subcore_axis_name="s")

    @functools.partial(
        pl.kernel,
        out_type=jax.ShapeDtypeStruct((2, n_out, half), jnp.float32),
        mesh=mesh,
        scratch_types=[
            pltpu.VMEM((2, 3, _K), jnp.int32),
            pltpu.VMEM((2, _K, half), jnp.float32),
            pltpu.VMEM_SHARED((n_out, half), jnp.float32),
            pltpu.SemaphoreType.DMA,
            pltpu.SemaphoreType.DMA,
            pltpu.SemaphoreType.DMA,
            pltpu.SemaphoreType.DMA,
        ],
        compiler_params=pltpu.CompilerParams(use_tc_tiling_on_sc=False,
                                             needs_layout_passes=False),
    )
    def spmm(x_hbm, edata_hbm, times_hbm, out_hbm,
             ebuf, rows, acc_sh, sem_i0, sem_i1, sem_g0, sem_g1):
        c = lax.axis_index("c")
        s = lax.axis_index("s")
        sem_i = (sem_i0, sem_i1)
        sem_g = (sem_g0, sem_g1)

        row0 = s * rpt
        nfull = rpt // _K
        rem = rpt % _K
        cbase = s * n_chunks

        pltpu.sync_copy(times_hbm, ebuf.at[0].at[0].at[pl.ds(0, _L)])
        times = ebuf[0, 0, pl.ds(0, _L)][0]

        def zero_acc():
            def zero_row(i, carry):
                for q in range(q_per_row):
                    rows[0, i, pl.ds(q * _L, _L)] = jnp.zeros((_L,),
                                                              jnp.float32)
                return carry
            lax.fori_loop(0, _K, zero_row, 0)
            for b in range(nfull):
                pltpu.sync_copy(rows.at[0],
                                acc_sh.at[pl.ds(row0 + b * _K, _K)])
            if rem:
                pltpu.sync_copy(rows.at[0].at[pl.ds(0, rem)],
                                acc_sh.at[pl.ds(row0 + nfull * _K, rem)])

        def issue_idx(j, p):
            pltpu.async_copy(edata_hbm.at[cbase + j], ebuf.at[p], sem_i[p])

        def wait_idx(p):
            pltpu.make_async_copy(edata_hbm.at[0], ebuf.at[p], sem_i[p]).wait()

        def scale_scatter(p, unrolled):
            rows_p = rows.at[p]

            def group(g, carry):
                av_bits = ebuf[p, 2, pl.ds(g * _L, _L)]
                av = plsc.bitcast(av_bits, jnp.float32)
                e0 = g * _L
                for i in range(_L):
                    a = av[i]
                    for q in range(q_per_row):
                        sl = pl.ds(q * _L, _L)
                        rows_p[e0 + i, sl] = rows_p[e0 + i, sl] * a
                return carry
            if unrolled:
                for g in range(_K // _L):
                    group(g, 0)
            else:
                lax.fori_loop(0, _K // _L, group, 0)
            pltpu.sync_copy(rows_p, acc_sh.at[ebuf.at[p].at[1]], add=True)

        def pipeline(src_hbm):
            def issue_gather(p):
                pltpu.async_copy(src_hbm.at[ebuf.at[p].at[0]], rows.at[p],
                                 sem_g[p])

            def wait_gather(p):
                pltpu.make_async_copy(src_hbm.at[pl.ds(0, _K)], rows.at[p],
                                      sem_g[p]).wait()

            issue_idx(0, 0)
            wait_idx(0)
            issue_gather(0)
            issue_idx(1, 1)

            def pair(jp, carry):
                j = 2 * jp
                wait_idx(1)
                issue_gather(1)
                wait_gather(0)
                scale_scatter(0, True)
                issue_idx(j + 2, 0)
                wait_idx(0)
                issue_gather(0)
                wait_gather(1)
                scale_scatter(1, True)
                issue_idx(j + 3, 1)
                return carry
            lax.fori_loop(0, n_chunks // 2 - 1, pair, 0)

            wait_idx(1)
            issue_gather(1)
            wait_gather(0)
            scale_scatter(0, False)
            wait_gather(1)
            scale_scatter(1, False)

        def writeback():
            for b in range(nfull):
                sl = pl.ds(row0 + b * _K, _K)
                pltpu.sync_copy(acc_sh.at[sl], out_hbm.at[c].at[sl])
            if rem:
                sl = pl.ds(row0 + nfull * _K, rem)
                pltpu.sync_copy(acc_sh.at[sl], out_hbm.at[c].at[sl])

        zero_acc()
        plsc.subcore_barrier()
        pipeline(x_hbm.at[c])
        plsc.subcore_barrier()
        writeback()
        plsc.subcore_barrier()

        def round_body(r, carry):
            zero_acc()
            plsc.subcore_barrier()
            pipeline(out_hbm.at[c])
            plsc.subcore_barrier()
            writeback()
            plsc.subcore_barrier()
            return carry
        lax.fori_loop(1, times, round_body, 0)

    return spmm


def kernel(features, adj_values, weight, edge_index, times):
    n, _ = features.shape
    m = weight.shape[1]
    half = m // 2
    e = edge_index.shape[1]

    src = edge_index[1].astype(jnp.int32)
    dst = edge_index[0].astype(jnp.int32)
    adj = adj_values.astype(jnp.float32)

    grain = _NS * 2 * _K
    e_pad = ((e + grain - 1) // grain) * grain
    if e_pad != e:
        pad = e_pad - e
        src = jnp.concatenate([src, jnp.zeros((pad,), jnp.int32)])
        dst = jnp.concatenate([dst, jnp.zeros((pad,), jnp.int32)])
        adj = jnp.concatenate([adj, jnp.zeros((pad,), jnp.float32)])

    n_chunks = e_pad // (_NS * _K)
    adj_bits = lax.bitcast_convert_type(adj, jnp.int32)
    edata = jnp.stack([src, dst, adj_bits])
    edata = edata.reshape(3, _NS, n_chunks, _K)
    edata = jnp.transpose(edata, (1, 2, 0, 3)).reshape(_NS * n_chunks, 3, _K)

    times_v = jnp.full((_L,), 1, jnp.int32) * times

    row_grain = _NS * 8
    n_pad = ((n + row_grain - 1) // row_grain) * row_grain

    support2 = _project_halves(features, weight)
    spmm = _make_spmm(n, n_pad, half, e_pad)
    out2 = spmm(support2, edata, times_v)
    return jnp.swapaxes(out2[:, :n, :], 0, 1).reshape(n, m)


# speedup vs baseline: 1.7885x; 1.0613x over previous
import functools

import jax
import jax.numpy as jnp
from jax import lax
from jax.experimental import pallas as pl
from jax.experimental.pallas import tpu as pltpu
from jax.experimental.pallas import tpu_sc as plsc

_NC = 2
_NS = 16
_L = 16
_K = 128


def _project_halves(features, weight, rows_per_block=2000):
    n, f = features.shape
    m = weight.shape[1]
    half = m // 2

    def body(f_ref, w_ref, o_ref):
        o_ref[0] = jnp.dot(f_ref[...], w_ref[0],
                           preferred_element_type=jnp.float32)

    w_halves = jnp.swapaxes(weight.reshape(f, 2, half), 0, 1)
    return pl.pallas_call(
        body,
        grid=(2, n // rows_per_block),
        in_specs=[
            pl.BlockSpec((rows_per_block, f), lambda c, r: (r, 0)),
            pl.BlockSpec((1, f, half), lambda c, r: (c, 0, 0)),
        ],
        out_specs=pl.BlockSpec((1, rows_per_block, half), lambda c, r: (c, r, 0)),
        out_shape=jax.ShapeDtypeStruct((2, n, half), jnp.float32),
    )(features, w_halves)


@functools.lru_cache
def _make_spmm(n_x, n_out, half, e_pad):
    ept = e_pad // _NS
    n_chunks = ept // _K
    rpt = n_out // _NS
    q_per_row = half // _L
    assert n_chunks % 2 == 0 and n_chunks >= 4

    mesh = plsc.VectorSubcoreMesh(core_axis_name="c", subcore_axis_name="s")

    @functools.partial(
        pl.kernel,
        out_type=jax.ShapeDtypeStruct((2, n_out, half), jnp.float32),
        mesh=mesh,
        scratch_types=[
            pltpu.VMEM((2, 3, _K), jnp.int32),
            pltpu.VMEM((2, _K, half), jnp.float32),
            pltpu.VMEM_SHARED((n_out, half), jnp.float32),
            pltpu.VMEM_SHARED((n_out, half), jnp.float32),
            pltpu.SemaphoreType.DMA,
            pltpu.SemaphoreType.DMA,
            pltpu.SemaphoreType.DMA,
            pltpu.SemaphoreType.DMA,
        ],
        compiler_params=pltpu.CompilerParams(use_tc_tiling_on_sc=False,
                                             needs_layout_passes=False),
    )
    def spmm(x_hbm, edata_hbm, times_hbm, out_hbm,
             ebuf, rows, acc_a, acc_b, sem_i0, sem_i1, sem_g0, sem_g1):
        c = lax.axis_index("c")
        s = lax.axis_index("s")
        sem_i = (sem_i0, sem_i1)
        sem_g = (sem_g0, sem_g1)

        row0 = s * rpt
        nfull = rpt // _K
        rem = rpt % _K
        cbase = s * n_chunks

        pltpu.sync_copy(times_hbm, ebuf.at[0].at[0].at[pl.ds(0, _L)])
        times = ebuf[0, 0, pl.ds(0, _L)][0]

        def zero_acc(acc_sh):
            def zero_row(i, carry):
                for q in range(q_per_row):
                    rows[0, i, pl.ds(q * _L, _L)] = jnp.zeros((_L,),
                                                              jnp.float32)
                return carry
            lax.fori_loop(0, _K, zero_row, 0)
            for b in range(nfull):
                pltpu.sync_copy(rows.at[0],
                                acc_sh.at[pl.ds(row0 + b * _K, _K)])
            if rem:
                pltpu.sync_copy(rows.at[0].at[pl.ds(0, rem)],
                                acc_sh.at[pl.ds(row0 + nfull * _K, rem)])

        def issue_idx(j, p):
            pltpu.async_copy(edata_hbm.at[cbase + j], ebuf.at[p], sem_i[p])

        def wait_idx(p):
            pltpu.make_async_copy(edata_hbm.at[0], ebuf.at[p], sem_i[p]).wait()

        def scale_scatter(p, unrolled, acc_sh):
            rows_p = rows.at[p]

            def group(g, carry):
                av_bits = ebuf[p, 2, pl.ds(g * _L, _L)]
                av = plsc.bitcast(av_bits, jnp.float32)
                e0 = g * _L
                for i in range(_L):
                    a = av[i]
                    for q in range(q_per_row):
                        sl = pl.ds(q * _L, _L)
                        rows_p[e0 + i, sl] = rows_p[e0 + i, sl] * a
                return carry
            if unrolled:
                for g in range(_K // _L):
                    group(g, 0)
            else:
                lax.fori_loop(0, _K // _L, group, 0)
            pltpu.sync_copy(rows_p, acc_sh.at[ebuf.at[p].at[1]], add=True)

        def pipeline(src_ref, acc_sh):
            def issue_gather(p):
                pltpu.async_copy(src_ref.at[ebuf.at[p].at[0]], rows.at[p],
                                 sem_g[p])

            def wait_gather(p):
                pltpu.make_async_copy(src_ref.at[pl.ds(0, _K)], rows.at[p],
                                      sem_g[p]).wait()

            issue_idx(0, 0)
            wait_idx(0)
            issue_gather(0)
            issue_idx(1, 1)

            def pair(jp, carry):
                j = 2 * jp
                wait_idx(1)
                issue_gather(1)
                wait_gather(0)
                scale_scatter(0, True, acc_sh)
                issue_idx(j + 2, 0)
                wait_idx(0)
                issue_gather(0)
                wait_gather(1)
                scale_scatter(1, True, acc_sh)
                issue_idx(j + 3, 1)
                return carry
            lax.fori_loop(0, n_chunks // 2 - 1, pair, 0)

            wait_idx(1)
            issue_gather(1)
            wait_gather(0)
            scale_scatter(0, False, acc_sh)
            wait_gather(1)
            scale_scatter(1, False, acc_sh)

        def writeback(acc_sh):
            for b in range(nfull):
                sl = pl.ds(row0 + b * _K, _K)
                pltpu.sync_copy(acc_sh.at[sl], out_hbm.at[c].at[sl])
            if rem:
                sl = pl.ds(row0 + nfull * _K, rem)
                pltpu.sync_copy(acc_sh.at[sl], out_hbm.at[c].at[sl])

        zero_acc(acc_a)
        plsc.subcore_barrier()
        pipeline(x_hbm.at[c], acc_a)
        plsc.subcore_barrier()

        def round_body(r, carry):
            odd = (r % 2) == 1

            @pl.when(odd)
            def _():
                zero_acc(acc_b)
                plsc.subcore_barrier()
                pipeline(acc_a, acc_b)
                plsc.subcore_barrier()

            @pl.when(jnp.logical_not(odd))
            def _():
                zero_acc(acc_a)
                plsc.subcore_barrier()
                pipeline(acc_b, acc_a)
                plsc.subcore_barrier()
            return carry
        lax.fori_loop(1, times, round_body, 0)

        @pl.when((times % 2) == 1)
        def _():
            writeback(acc_a)

        @pl.when((times % 2) == 0)
        def _():
            writeback(acc_b)

    return spmm


def kernel(features, adj_values, weight, edge_index, times):
    n, _ = features.shape
    m = weight.shape[1]
    half = m // 2
    e = edge_index.shape[1]

    src = edge_index[1].astype(jnp.int32)
    dst = edge_index[0].astype(jnp.int32)
    adj = adj_values.astype(jnp.float32)

    grain = _NS * 2 * _K
    e_pad = ((e + grain - 1) // grain) * grain
    if e_pad != e:
        pad = e_pad - e
        src = jnp.concatenate([src, jnp.zeros((pad,), jnp.int32)])
        dst = jnp.concatenate([dst, jnp.zeros((pad,), jnp.int32)])
        adj = jnp.concatenate([adj, jnp.zeros((pad,), jnp.float32)])

    n_chunks = e_pad // (_NS * _K)
    adj_bits = lax.bitcast_convert_type(adj, jnp.int32)
    edata = jnp.stack([src, dst, adj_bits])
    edata = edata.reshape(3, _NS, n_chunks, _K)
    edata = jnp.transpose(edata, (1, 2, 0, 3)).reshape(_NS * n_chunks, 3, _K)

    times_v = jnp.full((_L,), 1, jnp.int32) * times

    row_grain = _NS * 8
    n_pad = ((n + row_grain - 1) // row_grain) * row_grain

    support2 = _project_halves(features, weight)
    spmm = _make_spmm(n, n_pad, half, e_pad)
    out2 = spmm(support2, edata, times_v)
    return jnp.swapaxes(out2[:, :n, :], 0, 1).reshape(n, m)
